# Initial kernel scaffold; baseline (speedup 1.0000x reference)
import jax
import jax.numpy as jnp
from jax.experimental import pallas as pl


def kernel(imgs, c, seg, w, h, o):
    raise NotImplementedError("write your pallas kernel here")


# speedup vs baseline: 2.3331x; 2.3331x over previous
import functools

import jax
import jax.numpy as jnp
from jax.experimental import pallas as pl
from jax.experimental.pallas import tpu as pltpu


def _shift_max(x, r, axis):
    n = x.shape[axis]
    out = x
    for k in range(1, r + 1):
        if axis == 0:
            z = jnp.zeros((k, x.shape[1]), x.dtype)
            down = jnp.concatenate([z, x[: n - k, :]], axis=0)
            up = jnp.concatenate([x[k:, :], z], axis=0)
        else:
            z = jnp.zeros((x.shape[0], k), x.dtype)
            down = jnp.concatenate([z, x[:, : n - k]], axis=1)
            up = jnp.concatenate([x[:, k:], z], axis=1)
        out = jnp.maximum(out, jnp.maximum(down, up))
    return out


def _maxpool2d(x, r):
    return _shift_max(_shift_max(x, r, 0), r, 1)


def _fused_kernel(img_ref, c_ref, seg_ref, w_ref, h_ref, o_ref,
                  enh_ref, segment_ref, segbig_ref, confo_ref, confc_ref,
                  mnt_ref, conf_ref):
    img = img_ref[0]
    mi = jnp.min(img)
    mx = jnp.max(img)
    enh_ref[0] = (img - mi) / (mx - mi + 1e-6)

    seg5 = _maxpool2d(jnp.round(seg_ref[0]), 2)
    segment_ref[0] = seg5

    rep = (jax.lax.broadcasted_iota(jnp.int32, (512, 64), 0) // 8
           == jax.lax.broadcasted_iota(jnp.int32, (512, 64), 1)
           ).astype(jnp.float32)
    t = jax.lax.dot_general(seg5, rep, (((1,), (1,)), ((), ())),
                            preferred_element_type=jnp.float32)
    segbig_ref[0] = jax.lax.dot_general(
        rep, t, (((1,), (0,)), ((), ())),
        preferred_element_type=jnp.float32)

    cm = c_ref[0] * seg5
    local_max = _maxpool2d(cm, 7)
    keep = (cm >= local_max) & (cm > 0.45)
    confc_ref[0] = jnp.where(keep, cm, 0.0)
    confo_ref[0] = jnp.where(keep, o_ref[0], 0.0)
    mnt_ref[...] = jnp.sum(keep.astype(jnp.int32), axis=(0, 1),
                           keepdims=True)[None]

    conf_ref[0, 0] = c_ref[0]
    conf_ref[0, 1] = w_ref[0]
    conf_ref[0, 2] = h_ref[0]
    conf_ref[0, 3] = o_ref[0]


@functools.partial(jax.jit, static_argnames=())
def kernel(imgs, c, seg, w, h, o):
    B = imgs.shape[0]
    H, W = imgs.shape[2], imgs.shape[3]
    Hm, Wm = c.shape[2], c.shape[3]

    img3 = imgs.reshape(B, H, W)
    maps = [x.reshape(B, Hm, Wm) for x in (c, seg, w, h, o)]

    big_spec = pl.BlockSpec((1, H, W), lambda b: (b, 0, 0))
    map_spec = pl.BlockSpec((1, Hm, Wm), lambda b: (b, 0, 0))

    outs = pl.pallas_call(
        _fused_kernel,
        grid=(B,),
        in_specs=[big_spec] + [map_spec] * 5,
        out_specs=[
            big_spec,
            map_spec,
            big_spec,
            map_spec,
            map_spec,
            pl.BlockSpec((1, 1, 1), lambda b: (b, 0, 0)),
            pl.BlockSpec((1, 4, Hm, Wm), lambda b: (b, 0, 0, 0)),
        ],
        out_shape=[
            jax.ShapeDtypeStruct((B, H, W), jnp.float32),
            jax.ShapeDtypeStruct((B, Hm, Wm), jnp.float32),
            jax.ShapeDtypeStruct((B, H, W), jnp.float32),
            jax.ShapeDtypeStruct((B, Hm, Wm), jnp.float32),
            jax.ShapeDtypeStruct((B, Hm, Wm), jnp.float32),
            jax.ShapeDtypeStruct((B, 1, 1), jnp.int32),
            jax.ShapeDtypeStruct((B, 4, Hm, Wm), jnp.float32),
        ],
        compiler_params=pltpu.CompilerParams(
            dimension_semantics=("arbitrary",)),
    )(img3, *maps)

    enh, segment, segbig, confo, confc, mnt, conf = outs
    return (enh.reshape(B, 1, H, W),
            segment.reshape(B, 1, Hm, Wm),
            segbig.reshape(B, 1, H, W),
            confo.reshape(B, 1, Hm, Wm),
            confc.reshape(B, 1, Hm, Wm),
            mnt.reshape(B),
            conf)


# speedup vs baseline: 2.3677x; 1.0149x over previous
import functools

import jax
import jax.numpy as jnp
from jax.experimental import pallas as pl
from jax.experimental.pallas import tpu as pltpu


def _shift(x, k, axis):
    n = x.shape[axis]
    if axis == 0:
        z = jnp.zeros((abs(k), x.shape[1]), x.dtype)
        if k > 0:
            return jnp.concatenate([z, x[: n - k, :]], axis=0)
        return jnp.concatenate([x[-k:, :], z], axis=0)
    z = jnp.zeros((x.shape[0], abs(k)), x.dtype)
    if k > 0:
        return jnp.concatenate([z, x[:, : n - k]], axis=1)
    return jnp.concatenate([x[:, -k:], z], axis=1)


def _shift_max(x, r, axis):
    fwd, bwd = x, x
    covered = 1
    while covered < r + 1:
        s = min(covered, r + 1 - covered)
        fwd = jnp.maximum(fwd, _shift(fwd, -s, axis))
        bwd = jnp.maximum(bwd, _shift(bwd, s, axis))
        covered += s
    return jnp.maximum(fwd, bwd)


def _maxpool2d(x, r):
    return _shift_max(_shift_max(x, r, 0), r, 1)


def _fused_kernel(img_ref, c_ref, seg_ref, w_ref, h_ref, o_ref,
                  enh_ref, segment_ref, segbig_ref, confo_ref, confc_ref,
                  mnt_ref, conf_ref):
    img = img_ref[0]
    mi = jnp.min(img)
    mx = jnp.max(img)
    enh_ref[0] = (img - mi) / (mx - mi + 1e-6)

    seg5 = _maxpool2d((seg_ref[0] > 0.5).astype(jnp.float32), 2)
    segment_ref[0] = seg5

    rep = (jax.lax.broadcasted_iota(jnp.int32, (512, 64), 0) // 8
           == jax.lax.broadcasted_iota(jnp.int32, (512, 64), 1)
           ).astype(jnp.float32)
    t = jax.lax.dot_general(seg5, rep, (((1,), (1,)), ((), ())),
                            preferred_element_type=jnp.float32)
    segbig_ref[0] = jax.lax.dot_general(
        rep, t, (((1,), (0,)), ((), ())),
        preferred_element_type=jnp.float32)

    cm = c_ref[0] * seg5
    local_max = _maxpool2d(cm, 7)
    keep = (cm >= local_max) & (cm > 0.45)
    confc_ref[0] = jnp.where(keep, cm, 0.0)
    confo_ref[0] = jnp.where(keep, o_ref[0], 0.0)
    mnt_ref[...] = jnp.sum(keep.astype(jnp.int32), axis=(0, 1),
                           keepdims=True)[None]

    conf_ref[0, 0] = c_ref[0]
    conf_ref[0, 1] = w_ref[0]
    conf_ref[0, 2] = h_ref[0]
    conf_ref[0, 3] = o_ref[0]


@functools.partial(jax.jit, static_argnames=())
def kernel(imgs, c, seg, w, h, o):
    B = imgs.shape[0]
    H, W = imgs.shape[2], imgs.shape[3]
    Hm, Wm = c.shape[2], c.shape[3]

    img3 = imgs.reshape(B, H, W)
    maps = [x.reshape(B, Hm, Wm) for x in (c, seg, w, h, o)]

    big_spec = pl.BlockSpec((1, H, W), lambda b: (b, 0, 0))
    map_spec = pl.BlockSpec((1, Hm, Wm), lambda b: (b, 0, 0))

    outs = pl.pallas_call(
        _fused_kernel,
        grid=(B,),
        in_specs=[big_spec] + [map_spec] * 5,
        out_specs=[
            big_spec,
            map_spec,
            big_spec,
            map_spec,
            map_spec,
            pl.BlockSpec((1, 1, 1), lambda b: (b, 0, 0)),
            pl.BlockSpec((1, 4, Hm, Wm), lambda b: (b, 0, 0, 0)),
        ],
        out_shape=[
            jax.ShapeDtypeStruct((B, H, W), jnp.float32),
            jax.ShapeDtypeStruct((B, Hm, Wm), jnp.float32),
            jax.ShapeDtypeStruct((B, H, W), jnp.float32),
            jax.ShapeDtypeStruct((B, Hm, Wm), jnp.float32),
            jax.ShapeDtypeStruct((B, Hm, Wm), jnp.float32),
            jax.ShapeDtypeStruct((B, 1, 1), jnp.int32),
            jax.ShapeDtypeStruct((B, 4, Hm, Wm), jnp.float32),
        ],
        compiler_params=pltpu.CompilerParams(
            dimension_semantics=("arbitrary",)),
    )(img3, *maps)

    enh, segment, segbig, confo, confc, mnt, conf = outs
    return (enh.reshape(B, 1, H, W),
            segment.reshape(B, 1, Hm, Wm),
            segbig.reshape(B, 1, H, W),
            confo.reshape(B, 1, Hm, Wm),
            confc.reshape(B, 1, Hm, Wm),
            mnt.reshape(B),
            conf)
